# baseline (speedup 1.0000x reference)
import jax
import jax.numpy as jnp
from jax.experimental import pallas as pl
from jax.experimental.pallas import tpu as pltpu

B, N, F_IN, F_OUT = 2, 4096, 128, 128
BLK = 512


def _support_kernel(x_ref, w_ref, s_ref):
    w = w_ref[...].astype(jnp.bfloat16)
    for b in range(B):
        s = jnp.dot(x_ref[b].astype(jnp.bfloat16), w, preferred_element_type=jnp.float32)
        s_ref[:, b, :] = s.astype(jnp.bfloat16)


def _agg_kernel(adj_ref, s_ref, b_ref, o_ref):
    a = adj_ref[...].astype(jnp.bfloat16)
    r = jnp.dot(a, s_ref[...], preferred_element_type=jnp.float32)
    bias = b_ref[0]
    o_ref[0] = r[:, :F_OUT] + bias
    o_ref[1] = r[:, F_OUT:] + bias


def kernel(x, adj, weight, bias):
    support = pl.pallas_call(
        _support_kernel,
        grid=(1,),
        in_specs=[
            pl.BlockSpec((B, N, F_IN), lambda i: (0, 0, 0)),
            pl.BlockSpec((F_IN, F_OUT), lambda i: (0, 0)),
        ],
        out_specs=pl.BlockSpec((N, B, F_OUT), lambda i: (0, 0, 0)),
        out_shape=jax.ShapeDtypeStruct((N, B, F_OUT), jnp.bfloat16),
    )(x, weight)
    support = support.reshape(N, B * F_OUT)

    out = pl.pallas_call(
        _agg_kernel,
        grid=(N // BLK,),
        in_specs=[
            pl.BlockSpec((BLK, N), lambda i: (i, 0)),
            pl.BlockSpec((N, B * F_OUT), lambda i: (0, 0)),
            pl.BlockSpec((1, F_OUT), lambda i: (0, 0)),
        ],
        out_specs=pl.BlockSpec((B, BLK, F_OUT), lambda i: (0, i, 0)),
        out_shape=jax.ShapeDtypeStruct((B, N, F_OUT), jnp.float32),
        compiler_params=pltpu.CompilerParams(
            dimension_semantics=("parallel",),
        ),
    )(adj, support, bias.reshape(1, F_OUT))
    return out


# speedup vs baseline: 1.2110x; 1.2110x over previous
import jax
import jax.numpy as jnp
from jax.experimental import pallas as pl
from jax.experimental.pallas import tpu as pltpu

B, N, F_IN, F_OUT = 2, 4096, 128, 128
BLK = 256


def _gcn_kernel(adj_ref, x_ref, w_ref, b_ref, o_ref, s_ref):
    i = pl.program_id(0)

    @pl.when(i == 0)
    def _():
        w = w_ref[...].astype(jnp.bfloat16)
        s0 = jnp.dot(x_ref[0].astype(jnp.bfloat16), w, preferred_element_type=jnp.float32)
        s1 = jnp.dot(x_ref[1].astype(jnp.bfloat16), w, preferred_element_type=jnp.float32)
        s_ref[:, :F_OUT] = s0.astype(jnp.bfloat16)
        s_ref[:, F_OUT:] = s1.astype(jnp.bfloat16)

    a = adj_ref[...].astype(jnp.bfloat16)
    r = jnp.dot(a, s_ref[...], preferred_element_type=jnp.float32)
    bias = b_ref[0]
    o_ref[0] = r[:, :F_OUT] + bias
    o_ref[1] = r[:, F_OUT:] + bias


def kernel(x, adj, weight, bias):
    return pl.pallas_call(
        _gcn_kernel,
        grid=(N // BLK,),
        in_specs=[
            pl.BlockSpec((BLK, N), lambda i: (i, 0)),
            pl.BlockSpec((B, N, F_IN), lambda i: (0, 0, 0)),
            pl.BlockSpec((F_IN, F_OUT), lambda i: (0, 0)),
            pl.BlockSpec((1, F_OUT), lambda i: (0, 0)),
        ],
        out_specs=pl.BlockSpec((B, BLK, F_OUT), lambda i: (0, i, 0)),
        out_shape=jax.ShapeDtypeStruct((B, N, F_OUT), jnp.float32),
        scratch_shapes=[pltpu.VMEM((N, B * F_OUT), jnp.bfloat16)],
    )(adj, x, weight, bias.reshape(1, F_OUT))


# speedup vs baseline: 1.3870x; 1.1453x over previous
import jax
import jax.numpy as jnp
from jax.experimental import pallas as pl
from jax.experimental.pallas import tpu as pltpu

B, N, F_IN, F_OUT = 2, 4096, 128, 128
BLK = 512


def _gcn_kernel(adj_ref, x_ref, w_ref, b_ref, o_ref, s_ref):
    i = pl.program_id(0)

    @pl.when(i == 0)
    def _():
        w = w_ref[...]
        s_ref[:, :F_OUT] = jnp.dot(x_ref[0], w, preferred_element_type=jnp.float32)
        s_ref[:, F_OUT:] = jnp.dot(x_ref[1], w, preferred_element_type=jnp.float32)

    r = jnp.dot(adj_ref[...], s_ref[...], preferred_element_type=jnp.float32)
    bias = b_ref[0]
    o_ref[0] = r[:, :F_OUT] + bias
    o_ref[1] = r[:, F_OUT:] + bias


def kernel(x, adj, weight, bias):
    return pl.pallas_call(
        _gcn_kernel,
        grid=(N // BLK,),
        in_specs=[
            pl.BlockSpec((BLK, N), lambda i: (i, 0)),
            pl.BlockSpec((B, N, F_IN), lambda i: (0, 0, 0)),
            pl.BlockSpec((F_IN, F_OUT), lambda i: (0, 0)),
            pl.BlockSpec((1, F_OUT), lambda i: (0, 0)),
        ],
        out_specs=pl.BlockSpec((B, BLK, F_OUT), lambda i: (0, i, 0)),
        out_shape=jax.ShapeDtypeStruct((B, N, F_OUT), jnp.float32),
        scratch_shapes=[pltpu.VMEM((N, B * F_OUT), jnp.float32)],
    )(adj, x, weight, bias.reshape(1, F_OUT))
